# baseline (speedup 1.0000x reference)
import functools

import jax
import jax.numpy as jnp
from jax import lax
from jax.experimental import pallas as pl
from jax.experimental.pallas import tpu as pltpu
from jax.experimental.pallas import tpu_sc as plsc

N_POS = 1048576
N_LEV = 4
HASH_BITS = 21
MASK = (1 << HASH_BITS) - 1
P2 = 2654435761
INV_CS = (0.25, 0.125, 0.0625, 0.03125)
NC, NS = 2, 16
NW = NC * NS
B = 512
N_PER_TILE = N_POS // NW
N_CHUNKS = N_PER_TILE // B

_DNUMS = lax.GatherDimensionNumbers(
    offset_dims=(), collapsed_slice_dims=(0,), start_index_map=(0,))


def _take16(x, idx):
    return lax.gather(x, idx[:, None], _DNUMS, (1,),
                      mode=lax.GatherScatterMode.PROMISE_IN_BOUNDS)


def _rsqrt(v):
    i = lax.bitcast_convert_type(v, jnp.int32)
    y = lax.bitcast_convert_type(jnp.int32(0x5F3759DF) - (i >> 1), jnp.float32)
    h = 0.5 * v
    y = y * (1.5 - h * y * y)
    y = y * (1.5 - h * y * y)
    return y


_mesh = plsc.VectorSubcoreMesh(core_axis_name="c", subcore_axis_name="s")


@functools.partial(
    pl.kernel,
    out_type=jax.ShapeDtypeStruct((N_POS, 16), jnp.float32),
    mesh=_mesh,
    compiler_params=pltpu.CompilerParams(
        needs_layout_passes=False, use_tc_tiling_on_sc=False),
    scratch_types=[
        pltpu.VMEM((B, 2), jnp.float32),
        pltpu.VMEM((4 * B,), jnp.int32),
        pltpu.VMEM((4 * B,), jnp.int32),
        pltpu.VMEM((4 * B,), jnp.int32),
        pltpu.VMEM((4 * B,), jnp.int32),
        pltpu.VMEM((4 * B, 8), jnp.float32),
        pltpu.VMEM((4 * B, 8), jnp.float32),
        pltpu.VMEM((4 * B, 8), jnp.float32),
        pltpu.VMEM((4 * B, 8), jnp.float32),
        pltpu.VMEM((B, 16), jnp.float32),
        pltpu.VMEM((16,), jnp.float32),
        pltpu.SemaphoreType.DMA,
    ],
)
def _mrhg2d(pos_hbm, t0, t1, t2, t3, lw_hbm, out_hbm,
            pos_v, idx0, idx1, idx2, idx3, rows0, rows1, rows2, rows3,
            out_v, lw_v, sem):
    tables = (t0, t1, t2, t3)
    idxs = (idx0, idx1, idx2, idx3)
    rows = (rows0, rows1, rows2, rows3)
    wid = lax.axis_index("s") * NC + lax.axis_index("c")
    tile_base = wid * N_PER_TILE

    pltpu.sync_copy(lw_hbm, lw_v)
    lw_vec = lw_v[...]

    iota = lax.broadcasted_iota(jnp.int32, (16,), 0)
    lane4 = iota & 3
    grp4 = iota >> 2
    rot1 = (iota & ~3) | ((iota + 1) & 3)
    rot2 = (iota & ~3) | ((iota + 2) & 3)
    zeros = jnp.zeros((16,), jnp.int32)
    ones = jnp.ones((16,), jnp.int32)
    lane4p4 = lane4 + 4
    lw_splat = [_take16(lw_vec, jnp.full((16,), l, jnp.int32)) for l in range(N_LEV)]

    def chunk_body(c, _):
        base = tile_base + c * B
        pltpu.sync_copy(pos_hbm.at[pl.ds(base, B)], pos_v)

        def hash_body(j, _):
            row = j * 16 + iota
            px = plsc.load_gather(pos_v, [row, zeros])
            py = plsc.load_gather(pos_v, [row, ones])
            ix0 = (px * 0.25).astype(jnp.int32)
            iy0 = (py * 0.25).astype(jnp.int32)
            for l in range(N_LEV):
                ixu = (ix0 >> l).astype(jnp.uint32)
                iyu = (iy0 >> l).astype(jnp.uint32)
                t = iyu * jnp.uint32(P2)
                tn = t + jnp.uint32(P2)
                ixu1 = ixu + jnp.uint32(1)
                h00 = ((ixu ^ t) & jnp.uint32(MASK)).astype(jnp.int32)
                h10 = ((ixu1 ^ t) & jnp.uint32(MASK)).astype(jnp.int32)
                h01 = ((ixu ^ tn) & jnp.uint32(MASK)).astype(jnp.int32)
                h11 = ((ixu1 ^ tn) & jnp.uint32(MASK)).astype(jnp.int32)
                idxs[l][pl.ds(0 * B + j * 16, 16)] = h00 >> 1
                idxs[l][pl.ds(1 * B + j * 16, 16)] = h10 >> 1
                idxs[l][pl.ds(2 * B + j * 16, 16)] = h01 >> 1
                idxs[l][pl.ds(3 * B + j * 16, 16)] = h11 >> 1
            return 0

        lax.fori_loop(0, B // 16, hash_body, 0)

        copies = [
            pltpu.async_copy(tables[l].at[idxs[l]], rows[l], sem)
            for l in range(N_LEV)
        ]
        for cp in copies:
            cp.wait()

        def blend_body(j, _):
            row4 = j * 4 + grp4
            px = plsc.load_gather(pos_v, [row4, zeros])
            py = plsc.load_gather(pos_v, [row4, ones])
            for l in range(N_LEV):
                sx = px * INV_CS[l]
                sy = py * INV_CS[l]
                ixi = sx.astype(jnp.int32)
                iyi = sy.astype(jnp.int32)
                fx = sx - ixi.astype(jnp.float32)
                fy = sy - iyi.astype(jnp.float32)
                gx = 1.0 - fx
                gy = 1.0 - fy
                s = ((ixi ^ iyi) & 1) << 2
                c00 = s + lane4
                c10 = lane4p4 - s
                rv = rows[l]
                f00 = plsc.load_gather(rv, [0 * B + row4, c00])
                f10 = plsc.load_gather(rv, [1 * B + row4, c10])
                f01 = plsc.load_gather(rv, [2 * B + row4, c10])
                f11 = plsc.load_gather(rv, [3 * B + row4, c00])
                acc = (f00 * (gx * gy) + f10 * (fx * gy)
                       + f01 * (gx * fy) + f11 * (fx * fy))
                s1 = acc + _take16(acc, rot1)
                s2 = s1 + _take16(s1, rot2)
                mu = s2 * 0.25
                d = acc - mu
                sq = d * d
                v1 = sq + _take16(sq, rot1)
                v2 = v1 + _take16(v1, rot2)
                inv = _rsqrt(v2 * 0.25 + 1e-5)
                plsc.store_scatter(out_v, [row4, l * 4 + lane4], d * inv * lw_splat[l])
            return 0

        lax.fori_loop(0, B // 4, blend_body, 0)

        pltpu.sync_copy(out_v, out_hbm.at[pl.ds(base, B)])
        return 0

    lax.fori_loop(0, N_CHUNKS, chunk_body, 0)


def kernel(positions, table0, table1, table2, table3, level_weights):
    lw16 = jnp.zeros((16,), jnp.float32).at[:4].set(level_weights)
    return _mrhg2d(positions,
                   table0.reshape(-1, 8), table1.reshape(-1, 8),
                   table2.reshape(-1, 8), table3.reshape(-1, 8), lw16)


# speedup vs baseline: 4.3434x; 4.3434x over previous
import functools

import jax
import jax.numpy as jnp
from jax import lax
from jax.experimental import pallas as pl
from jax.experimental.pallas import tpu as pltpu
from jax.experimental.pallas import tpu_sc as plsc

N_POS = 1048576
N_LEV = 4
HASH_BITS = 21
MASK = (1 << HASH_BITS) - 1
P2 = 2654435761
INV_CS = (0.25, 0.125, 0.0625, 0.03125)
NC, NS = 2, 16
NW = NC * NS
B = 512
N_PER_TILE = N_POS // NW
N_CHUNKS = N_PER_TILE // B
NBLK = 1 << (HASH_BITS - 7)
BLK_PER_W = NBLK // NW
TBLK = 64

_DNUMS = lax.GatherDimensionNumbers(
    offset_dims=(), collapsed_slice_dims=(0,), start_index_map=(0,))


def _take16(x, idx):
    return lax.gather(x, idx[:, None], _DNUMS, (1,),
                      mode=lax.GatherScatterMode.PROMISE_IN_BOUNDS)


def _rsqrt(v):
    i = lax.bitcast_convert_type(v, jnp.int32)
    y = lax.bitcast_convert_type(jnp.int32(0x5F3759DF) - (i >> 1), jnp.float32)
    h = 0.5 * v
    y = y * (1.5 - h * y * y)
    y = y * (1.5 - h * y * y)
    return y


_mesh = plsc.VectorSubcoreMesh(core_axis_name="c", subcore_axis_name="s")
_params = pltpu.CompilerParams(
    needs_layout_passes=False, use_tc_tiling_on_sc=False)


@functools.partial(
    pl.kernel,
    out_type=[jax.ShapeDtypeStruct((1 << (HASH_BITS - 1), 8), jnp.float32)
              for _ in range(N_LEV)],
    mesh=_mesh,
    compiler_params=_params,
    scratch_types=[
        pltpu.VMEM((TBLK * 512,), jnp.float32),
        pltpu.VMEM((TBLK * 64, 8), jnp.float32),
    ],
)
def _transpose(x0, x1, x2, x3, o0, o1, o2, o3, in_v, out_v):
    xs = (x0, x1, x2, x3)
    outs = (o0, o1, o2, o3)
    wid = lax.axis_index("s") * NC + lax.axis_index("c")

    iota = lax.broadcasted_iota(jnp.int32, (16,), 0)
    lane4 = iota & 3
    grp4 = iota >> 2
    grp8 = iota >> 3
    lane8 = iota & 7
    gpat = lane4 * 128 + grp4

    for t in range(N_LEV):
        def mc_body(m, _, t=t):
            blk0 = wid * BLK_PER_W + m * TBLK
            pltpu.sync_copy(xs[t].at[pl.ds(blk0 * 512, TBLK * 512)], in_v)

            def blk_body(b, _):
                for v in range(32):
                    vec = plsc.load_gather(in_v, [b * 512 + 4 * v + gpat])
                    plsc.store_scatter(out_v, [b * 64 + 2 * v + grp8, lane8], vec)
                return 0

            lax.fori_loop(0, TBLK, blk_body, 0)
            pltpu.sync_copy(out_v, outs[t].at[pl.ds(blk0 * 64, TBLK * 64)])
            return 0

        lax.fori_loop(0, BLK_PER_W // TBLK, mc_body, 0)


@functools.partial(
    pl.kernel,
    out_type=jax.ShapeDtypeStruct((N_POS * 16,), jnp.float32),
    mesh=_mesh,
    compiler_params=_params,
    scratch_types=[
        pltpu.VMEM((2 * B,), jnp.float32),
        pltpu.VMEM((4 * B,), jnp.int32),
        pltpu.VMEM((4 * B,), jnp.int32),
        pltpu.VMEM((4 * B,), jnp.int32),
        pltpu.VMEM((4 * B,), jnp.int32),
        pltpu.VMEM((4 * B, 8), jnp.float32),
        pltpu.VMEM((4 * B, 8), jnp.float32),
        pltpu.VMEM((4 * B, 8), jnp.float32),
        pltpu.VMEM((4 * B, 8), jnp.float32),
        pltpu.VMEM((16 * B,), jnp.float32),
        pltpu.VMEM((16,), jnp.float32),
        pltpu.SemaphoreType.DMA,
    ],
)
def _mrhg2d(pos_hbm, t0, t1, t2, t3, lw_hbm, out_hbm,
            pos_v, idx0, idx1, idx2, idx3, rows0, rows1, rows2, rows3,
            out_v, lw_v, sem):
    tables = (t0, t1, t2, t3)
    idxs = (idx0, idx1, idx2, idx3)
    rows = (rows0, rows1, rows2, rows3)
    wid = lax.axis_index("s") * NC + lax.axis_index("c")
    tile_base = wid * N_PER_TILE

    pltpu.sync_copy(lw_hbm, lw_v)
    lw_vec = lw_v[...]

    iota = lax.broadcasted_iota(jnp.int32, (16,), 0)
    lane4 = iota & 3
    grp4 = iota >> 2
    rot1 = (iota & ~3) | ((iota + 1) & 3)
    rot2 = (iota & ~3) | ((iota + 2) & 3)
    lane4p4 = lane4 + 4
    outpat = grp4 * 16 + lane4
    lw_splat = [_take16(lw_vec, jnp.full((16,), l, jnp.int32)) for l in range(N_LEV)]

    def chunk_body(c, _):
        base = tile_base + c * B
        pltpu.sync_copy(pos_hbm.at[pl.ds(2 * base, 2 * B)], pos_v)

        def hash_body(j, _):
            xoff = (j >> 3) * 256 + (j & 7) * 16 + iota
            px = plsc.load_gather(pos_v, [xoff])
            py = plsc.load_gather(pos_v, [xoff + 128])
            ix0 = (px * 0.25).astype(jnp.int32)
            iy0 = (py * 0.25).astype(jnp.int32)
            for l in range(N_LEV):
                ixu = (ix0 >> l).astype(jnp.uint32)
                iyu = (iy0 >> l).astype(jnp.uint32)
                t = iyu * jnp.uint32(P2)
                tn = t + jnp.uint32(P2)
                ixu1 = ixu + jnp.uint32(1)
                h00 = ((ixu ^ t) & jnp.uint32(MASK)).astype(jnp.int32)
                h10 = ((ixu1 ^ t) & jnp.uint32(MASK)).astype(jnp.int32)
                h01 = ((ixu ^ tn) & jnp.uint32(MASK)).astype(jnp.int32)
                h11 = ((ixu1 ^ tn) & jnp.uint32(MASK)).astype(jnp.int32)
                idxs[l][pl.ds(0 * B + j * 16, 16)] = h00 >> 1
                idxs[l][pl.ds(1 * B + j * 16, 16)] = h10 >> 1
                idxs[l][pl.ds(2 * B + j * 16, 16)] = h01 >> 1
                idxs[l][pl.ds(3 * B + j * 16, 16)] = h11 >> 1
            return 0

        lax.fori_loop(0, B // 16, hash_body, 0)

        copies = [
            pltpu.async_copy(tables[l].at[idxs[l]], rows[l], sem)
            for l in range(N_LEV)
        ]
        for cp in copies:
            cp.wait()

        def blend_body(j, _):
            row4 = j * 4 + grp4
            xoff = (j >> 5) * 256 + (j & 31) * 4 + grp4
            px = plsc.load_gather(pos_v, [xoff])
            py = plsc.load_gather(pos_v, [xoff + 128])
            for l in range(N_LEV):
                sx = px * INV_CS[l]
                sy = py * INV_CS[l]
                ixi = sx.astype(jnp.int32)
                iyi = sy.astype(jnp.int32)
                fx = sx - ixi.astype(jnp.float32)
                fy = sy - iyi.astype(jnp.float32)
                gx = 1.0 - fx
                gy = 1.0 - fy
                s = ((ixi ^ iyi) & 1) << 2
                c00 = s + lane4
                c10 = lane4p4 - s
                rv = rows[l]
                f00 = plsc.load_gather(rv, [0 * B + row4, c00])
                f10 = plsc.load_gather(rv, [1 * B + row4, c10])
                f01 = plsc.load_gather(rv, [2 * B + row4, c10])
                f11 = plsc.load_gather(rv, [3 * B + row4, c00])
                acc = (f00 * (gx * gy) + f10 * (fx * gy)
                       + f01 * (gx * fy) + f11 * (fx * fy))
                s1 = acc + _take16(acc, rot1)
                s2 = s1 + _take16(s1, rot2)
                mu = s2 * 0.25
                d = acc - mu
                sq = d * d
                v1 = sq + _take16(sq, rot1)
                v2 = v1 + _take16(v1, rot2)
                inv = _rsqrt(v2 * 0.25 + 1e-5)
                plsc.store_scatter(out_v, [j * 64 + l * 4 + outpat],
                                   d * inv * lw_splat[l])
            return 0

        lax.fori_loop(0, B // 4, blend_body, 0)

        pltpu.sync_copy(out_v, out_hbm.at[pl.ds(16 * base, 16 * B)])
        return 0

    lax.fori_loop(0, N_CHUNKS, chunk_body, 0)


def kernel(positions, table0, table1, table2, table3, level_weights):
    xs = [t.reshape(NBLK, 128, 4).transpose(0, 2, 1).reshape(-1)
          for t in (table0, table1, table2, table3)]
    t8s = _transpose(*xs)
    pos1d = positions.reshape(N_POS // 128, 128, 2).transpose(0, 2, 1).reshape(-1)
    lw16 = jnp.zeros((16,), jnp.float32).at[:4].set(level_weights)
    out = _mrhg2d(pos1d, *t8s, lw16)
    return out.reshape(N_POS, 16)


# speedup vs baseline: 5.9540x; 1.3708x over previous
import functools

import jax
import jax.numpy as jnp
from jax import lax
from jax.experimental import pallas as pl
from jax.experimental.pallas import tpu as pltpu
from jax.experimental.pallas import tpu_sc as plsc

N_POS = 1048576
N_LEV = 4
HASH_BITS = 21
MASK = (1 << HASH_BITS) - 1
P2 = 2654435761
INV_CS = (0.25, 0.125, 0.0625, 0.03125)
NC, NS = 2, 16
NW = NC * NS
B = 256
N_PER_TILE = N_POS // NW
N_CHUNKS = N_PER_TILE // B
NBLK = 1 << (HASH_BITS - 7)
BLK_PER_W = NBLK // NW
TBLK = 64

_DNUMS = lax.GatherDimensionNumbers(
    offset_dims=(), collapsed_slice_dims=(0,), start_index_map=(0,))


def _take16(x, idx):
    return lax.gather(x, idx[:, None], _DNUMS, (1,),
                      mode=lax.GatherScatterMode.PROMISE_IN_BOUNDS)


def _rsqrt(v):
    i = lax.bitcast_convert_type(v, jnp.int32)
    y = lax.bitcast_convert_type(jnp.int32(0x5F3759DF) - (i >> 1), jnp.float32)
    y = y * (1.5 - (0.5 * v) * y * y)
    return y


_mesh = plsc.VectorSubcoreMesh(core_axis_name="c", subcore_axis_name="s")
_params = pltpu.CompilerParams(
    needs_layout_passes=False, use_tc_tiling_on_sc=False)


@functools.partial(
    pl.kernel,
    out_type=[jax.ShapeDtypeStruct((1 << (HASH_BITS - 1), 8), jnp.float32)
              for _ in range(N_LEV)],
    mesh=_mesh,
    compiler_params=_params,
    scratch_types=[
        pltpu.VMEM((TBLK * 512,), jnp.float32),
        pltpu.VMEM((TBLK * 64, 8), jnp.float32),
    ],
)
def _transpose(x0, x1, x2, x3, o0, o1, o2, o3, in_v, out_v):
    xs = (x0, x1, x2, x3)
    outs = (o0, o1, o2, o3)
    wid = lax.axis_index("s") * NC + lax.axis_index("c")

    iota = lax.broadcasted_iota(jnp.int32, (16,), 0)
    lane4 = iota & 3
    grp4 = iota >> 2
    grp8 = iota >> 3
    lane8 = iota & 7
    gpat = lane4 * 128 + grp4

    for t in range(N_LEV):
        def mc_body(m, _, t=t):
            blk0 = wid * BLK_PER_W + m * TBLK
            pltpu.sync_copy(xs[t].at[pl.ds(blk0 * 512, TBLK * 512)], in_v)

            def blk_body(b, _):
                for v in range(32):
                    vec = plsc.load_gather(in_v, [b * 512 + 4 * v + gpat])
                    plsc.store_scatter(out_v, [b * 64 + 2 * v + grp8, lane8], vec)
                return 0

            lax.fori_loop(0, TBLK, blk_body, 0)
            pltpu.sync_copy(out_v, outs[t].at[pl.ds(blk0 * 64, TBLK * 64)])
            return 0

        lax.fori_loop(0, BLK_PER_W // TBLK, mc_body, 0)


@functools.partial(
    pl.kernel,
    out_type=jax.ShapeDtypeStruct((N_POS * 16,), jnp.float32),
    mesh=_mesh,
    compiler_params=_params,
    scratch_types=(
        [pltpu.VMEM((2 * B,), jnp.float32) for _ in range(2)]
        + [pltpu.VMEM((4 * B,), jnp.int32) for _ in range(8)]
        + [pltpu.VMEM((4 * B, 8), jnp.float32) for _ in range(8)]
        + [pltpu.VMEM((16 * B,), jnp.float32) for _ in range(2)]
        + [pltpu.VMEM((16,), jnp.float32)]
        + [pltpu.SemaphoreType.DMA for _ in range(6)]
    ),
)
def _mrhg2d(pos_hbm, t0, t1, t2, t3, lw_hbm, out_hbm,
            pv0, pv1, i00, i01, i10, i11, i20, i21, i30, i31,
            r00, r01, r10, r11, r20, r21, r30, r31,
            ov0, ov1, lw_v, ts0, ts1, ps0, ps1, os0, os1):
    tables = (t0, t1, t2, t3)
    posv = (pv0, pv1)
    idxs = ((i00, i01), (i10, i11), (i20, i21), (i30, i31))
    rows = ((r00, r01), (r10, r11), (r20, r21), (r30, r31))
    outv = (ov0, ov1)
    tsem = (ts0, ts1)
    psem = (ps0, ps1)
    osem = (os0, os1)
    wid = lax.axis_index("s") * NC + lax.axis_index("c")
    tile_base = wid * N_PER_TILE

    pltpu.sync_copy(lw_hbm, lw_v)
    lw_vec = lw_v[...]

    iota = lax.broadcasted_iota(jnp.int32, (16,), 0)
    lane4 = iota & 3
    grp4 = iota >> 2
    rot1 = (iota & ~3) | ((iota + 1) & 3)
    rot2 = (iota & ~3) | ((iota + 2) & 3)
    lane4p4 = lane4 + 4
    outpat = grp4 * 16 + lane4
    lw_splat = [_take16(lw_vec, jnp.full((16,), l, jnp.int32)) for l in range(N_LEV)]

    def pos_slice(c):
        return pos_hbm.at[pl.ds(2 * (tile_base + c * B), 2 * B)]

    def out_slice(c):
        return out_hbm.at[pl.ds(16 * (tile_base + c * B), 16 * B)]

    def do_hash(c, pb):
        def hash_body(j, _):
            xoff = (j >> 3) * 256 + (j & 7) * 16 + iota
            px = plsc.load_gather(posv[pb], [xoff])
            py = plsc.load_gather(posv[pb], [xoff + 128])
            ix0 = (px * 0.25).astype(jnp.int32)
            iy0 = (py * 0.25).astype(jnp.int32)
            for l in range(N_LEV):
                ixu = (ix0 >> l).astype(jnp.uint32)
                iyu = (iy0 >> l).astype(jnp.uint32)
                t = iyu * jnp.uint32(P2)
                tn = t + jnp.uint32(P2)
                ixu1 = ixu + jnp.uint32(1)
                h00 = ((ixu ^ t) & jnp.uint32(MASK)).astype(jnp.int32)
                h10 = ((ixu1 ^ t) & jnp.uint32(MASK)).astype(jnp.int32)
                h01 = ((ixu ^ tn) & jnp.uint32(MASK)).astype(jnp.int32)
                h11 = ((ixu1 ^ tn) & jnp.uint32(MASK)).astype(jnp.int32)
                idxs[l][pb][pl.ds(0 * B + j * 16, 16)] = h00 >> 1
                idxs[l][pb][pl.ds(1 * B + j * 16, 16)] = h10 >> 1
                idxs[l][pb][pl.ds(2 * B + j * 16, 16)] = h01 >> 1
                idxs[l][pb][pl.ds(3 * B + j * 16, 16)] = h11 >> 1
            return 0

        lax.fori_loop(0, B // 16, hash_body, 0)

    def fire_gathers(b):
        for l in range(N_LEV):
            pltpu.async_copy(tables[l].at[idxs[l][b]], rows[l][b], tsem[b])

    def wait_gathers(b):
        for l in range(N_LEV):
            pltpu.make_async_copy(
                tables[l].at[idxs[l][b]], rows[l][b], tsem[b]).wait()

    def do_blend(c, b):
        def blend_body(j, _):
            row4 = j * 4 + grp4
            xoff = (j >> 5) * 256 + (j & 31) * 4 + grp4
            px = plsc.load_gather(posv[b], [xoff])
            py = plsc.load_gather(posv[b], [xoff + 128])
            for l in range(N_LEV):
                sx = px * INV_CS[l]
                sy = py * INV_CS[l]
                ixi = sx.astype(jnp.int32)
                iyi = sy.astype(jnp.int32)
                fx = sx - ixi.astype(jnp.float32)
                fy = sy - iyi.astype(jnp.float32)
                gx = 1.0 - fx
                gy = 1.0 - fy
                s = ((ixi ^ iyi) & 1) << 2
                c00 = s + lane4
                c10 = lane4p4 - s
                rv = rows[l][b]
                f00 = plsc.load_gather(rv, [0 * B + row4, c00])
                f10 = plsc.load_gather(rv, [1 * B + row4, c10])
                f01 = plsc.load_gather(rv, [2 * B + row4, c10])
                f11 = plsc.load_gather(rv, [3 * B + row4, c00])
                acc = (f00 * (gx * gy) + f10 * (fx * gy)
                       + f01 * (gx * fy) + f11 * (fx * fy))
                s1 = acc + _take16(acc, rot1)
                s2 = s1 + _take16(s1, rot2)
                mu = s2 * 0.25
                d = acc - mu
                sq = d * d
                v1 = sq + _take16(sq, rot1)
                v2 = v1 + _take16(v1, rot2)
                inv = _rsqrt(v2 * 0.25 + 1e-5)
                plsc.store_scatter(outv[b], [j * 64 + l * 4 + outpat],
                                   d * inv * lw_splat[l])
            return 0

        lax.fori_loop(0, B // 4, blend_body, 0)

    pltpu.sync_copy(pos_slice(0), posv[0])
    do_hash(0, 0)
    fire_gathers(0)
    pltpu.async_copy(pos_slice(1), posv[1], psem[1])

    def pair_body(p, _):
        for b in (0, 1):
            i = 2 * p + b

            @pl.when(i + 1 < N_CHUNKS)
            def _(b=b, i=i):
                pltpu.make_async_copy(pos_slice(0), posv[b ^ 1],
                                      psem[b ^ 1]).wait()
                do_hash(i + 1, b ^ 1)
                fire_gathers(b ^ 1)

            wait_gathers(b)

            @pl.when(i >= 2)
            def _(b=b, i=i):
                pltpu.make_async_copy(outv[b], out_slice(0), osem[b]).wait()

            do_blend(i, b)
            pltpu.async_copy(outv[b], out_slice(i), osem[b])

            @pl.when(i + 2 < N_CHUNKS)
            def _(b=b, i=i):
                pltpu.async_copy(pos_slice(i + 2), posv[b], psem[b])
        return 0

    lax.fori_loop(0, N_CHUNKS // 2, pair_body, 0)
    pltpu.make_async_copy(outv[0], out_slice(0), osem[0]).wait()
    pltpu.make_async_copy(outv[1], out_slice(0), osem[1]).wait()


def kernel(positions, table0, table1, table2, table3, level_weights):
    xs = [t.reshape(NBLK, 128, 4).transpose(0, 2, 1).reshape(-1)
          for t in (table0, table1, table2, table3)]
    t8s = _transpose(*xs)
    pos1d = positions.reshape(N_POS // 128, 128, 2).transpose(0, 2, 1).reshape(-1)
    lw16 = jnp.zeros((16,), jnp.float32).at[:4].set(level_weights)
    out = _mrhg2d(pos1d, *t8s, lw16)
    return out.reshape(N_POS, 16)


# speedup vs baseline: 6.5022x; 1.0921x over previous
import functools

import jax
import jax.numpy as jnp
from jax import lax
from jax.experimental import pallas as pl
from jax.experimental.pallas import tpu as pltpu
from jax.experimental.pallas import tpu_sc as plsc

N_POS = 1048576
HASH_BITS = 21
MASK = (1 << HASH_BITS) - 1
P2 = 2654435761
NC, NS = 2, 16
NW = NC * NS
B = 256
N_PER_TILE = N_POS // NW
N_CHUNKS = N_PER_TILE // B
NBLK = 1 << (HASH_BITS - 7)
BLK_PER_W = NBLK // NW
TBLK = 64

GRID_NG = (129, 65, 33)
FILL_BATCH = 512


def _batches(ng):
    ncells = ng * ng
    full = ncells // FILL_BATCH
    tail = ncells - full * FILL_BATCH
    tail_pad = -(-tail // 16) * 16
    out = [(i * FILL_BATCH, FILL_BATCH) for i in range(full)]
    if tail_pad:
        out.append((full * FILL_BATCH, tail_pad))
    return out


GRID_BATCHES = [_batches(ng) for ng in GRID_NG]
GRID_WORDS = [(bs[-1][0] + bs[-1][1]) * 4 for bs in GRID_BATCHES]
TAIL_SIZES = sorted({n * 4 for bs in GRID_BATCHES for _, n in bs if n != FILL_BATCH})

_DNUMS = lax.GatherDimensionNumbers(
    offset_dims=(), collapsed_slice_dims=(0,), start_index_map=(0,))


def _take16(x, idx):
    return lax.gather(x, idx[:, None], _DNUMS, (1,),
                      mode=lax.GatherScatterMode.PROMISE_IN_BOUNDS)


def _rsqrt(v):
    i = lax.bitcast_convert_type(v, jnp.int32)
    y = lax.bitcast_convert_type(jnp.int32(0x5F3759DF) - (i >> 1), jnp.float32)
    y = y * (1.5 - (0.5 * v) * y * y)
    return y


_mesh = plsc.VectorSubcoreMesh(core_axis_name="c", subcore_axis_name="s")
_params = pltpu.CompilerParams(
    needs_layout_passes=False, use_tc_tiling_on_sc=False)


@functools.partial(
    pl.kernel,
    out_type=jax.ShapeDtypeStruct((1 << (HASH_BITS - 1), 8), jnp.float32),
    mesh=_mesh,
    compiler_params=_params,
    scratch_types=[
        pltpu.VMEM((TBLK * 512,), jnp.float32),
        pltpu.VMEM((TBLK * 64, 8), jnp.float32),
    ],
)
def _transpose(x0, o0, in_v, out_v):
    wid = lax.axis_index("s") * NC + lax.axis_index("c")

    iota = lax.broadcasted_iota(jnp.int32, (16,), 0)
    lane4 = iota & 3
    grp4 = iota >> 2
    grp8 = iota >> 3
    lane8 = iota & 7
    gpat = lane4 * 128 + grp4

    def mc_body(m, _):
        blk0 = wid * BLK_PER_W + m * TBLK
        pltpu.sync_copy(x0.at[pl.ds(blk0 * 512, TBLK * 512)], in_v)

        def blk_body(b, _):
            for v in range(32):
                vec = plsc.load_gather(in_v, [b * 512 + 4 * v + gpat])
                plsc.store_scatter(out_v, [b * 64 + 2 * v + grp8, lane8], vec)
            return 0

        lax.fori_loop(0, TBLK, blk_body, 0)
        pltpu.sync_copy(out_v, o0.at[pl.ds(blk0 * 64, TBLK * 64)])
        return 0

    lax.fori_loop(0, BLK_PER_W // TBLK, mc_body, 0)


@functools.partial(
    pl.kernel,
    out_type=jax.ShapeDtypeStruct((N_POS * 16,), jnp.float32),
    mesh=_mesh,
    compiler_params=_params,
    scratch_types=(
        [pltpu.VMEM((2 * B,), jnp.float32) for _ in range(2)]
        + [pltpu.VMEM((4 * B,), jnp.int32) for _ in range(2)]
        + [pltpu.VMEM((4 * B, 8), jnp.float32) for _ in range(2)]
        + [pltpu.VMEM((16 * B,), jnp.float32) for _ in range(2)]
        + [pltpu.VMEM((16,), jnp.float32)]
        + [pltpu.VMEM((w,), jnp.float32) for w in GRID_WORDS]
        + [pltpu.VMEM((FILL_BATCH * 4,), jnp.int32)]
        + [pltpu.VMEM((w,), jnp.int32) for w in TAIL_SIZES]
        + [pltpu.SemaphoreType.DMA for _ in range(7)]
    ),
)
def _mrhg2d(pos_hbm, t8, x1, x2, x3, lw_hbm, out_hbm,
            pv0, pv1, i0a, i0b, r0a, r0b, ov0, ov1, lw_v,
            g1, g2, g3, fidx, *rest):
    tails = rest[:len(TAIL_SIZES)]
    ts0, ts1, ps0, ps1, os0, os1, fsem = rest[len(TAIL_SIZES):]
    xs = (x1, x2, x3)
    grids = (g1, g2, g3)
    posv = (pv0, pv1)
    idx0 = (i0a, i0b)
    rows0 = (r0a, r0b)
    outv = (ov0, ov1)
    tsem = (ts0, ts1)
    psem = (ps0, ps1)
    osem = (os0, os1)
    wid = lax.axis_index("s") * NC + lax.axis_index("c")
    tile_base = wid * N_PER_TILE

    pltpu.sync_copy(lw_hbm, lw_v)
    lw_vec = lw_v[...]

    iota = lax.broadcasted_iota(jnp.int32, (16,), 0)
    lane4 = iota & 3
    grp4 = iota >> 2
    rot1 = (iota & ~3) | ((iota + 1) & 3)
    rot2 = (iota & ~3) | ((iota + 2) & 3)
    lane4p4 = lane4 + 4
    lane128 = lane4 * 128
    outpat = grp4 * 16 + lane4
    lw_splat = [_take16(lw_vec, jnp.full((16,), l, jnp.int32)) for l in range(4)]

    for g in range(3):
        ng = GRID_NG[g]
        inv_ng = jnp.float32(1.0 / ng)
        for off, n in GRID_BATCHES[g]:
            buf = fidx if n == FILL_BATCH else tails[TAIL_SIZES.index(n * 4)]

            def fill_gen(j, _, off=off, ng=ng, inv_ng=inv_ng, buf=buf):
                k = off + j * 16 + iota
                gx = (k.astype(jnp.float32) * inv_ng + 0.004).astype(jnp.int32)
                gy = k - gx * ng
                h = ((gx.astype(jnp.uint32)
                      ^ (gy.astype(jnp.uint32) * jnp.uint32(P2)))
                     & jnp.uint32(MASK)).astype(jnp.int32)
                base = ((h >> 7) << 9) + (h & 127)
                for q in range(4):
                    rep = _take16(base, 4 * q + grp4)
                    buf[pl.ds((j * 4 + q) * 16, 16)] = rep + lane128
                return 0

            lax.fori_loop(0, n // 16, fill_gen, 0)
            pltpu.async_copy(xs[g].at[buf], grids[g].at[pl.ds(off * 4, n * 4)],
                             fsem)
            pltpu.make_async_copy(xs[g].at[buf],
                                  grids[g].at[pl.ds(off * 4, n * 4)],
                                  fsem).wait()

    def pos_slice(c):
        return pos_hbm.at[pl.ds(2 * (tile_base + c * B), 2 * B)]

    def out_slice(c):
        return out_hbm.at[pl.ds(16 * (tile_base + c * B), 16 * B)]

    def do_hash(c, pb):
        def hash_body(j, _):
            xoff = (j >> 3) * 256 + (j & 7) * 16 + iota
            px = plsc.load_gather(posv[pb], [xoff])
            py = plsc.load_gather(posv[pb], [xoff + 128])
            ixu = (px * 0.25).astype(jnp.int32).astype(jnp.uint32)
            iyu = (py * 0.25).astype(jnp.int32).astype(jnp.uint32)
            t = iyu * jnp.uint32(P2)
            tn = t + jnp.uint32(P2)
            ixu1 = ixu + jnp.uint32(1)
            h00 = ((ixu ^ t) & jnp.uint32(MASK)).astype(jnp.int32)
            h10 = ((ixu1 ^ t) & jnp.uint32(MASK)).astype(jnp.int32)
            h01 = ((ixu ^ tn) & jnp.uint32(MASK)).astype(jnp.int32)
            h11 = ((ixu1 ^ tn) & jnp.uint32(MASK)).astype(jnp.int32)
            idx0[pb][pl.ds(0 * B + j * 16, 16)] = h00 >> 1
            idx0[pb][pl.ds(1 * B + j * 16, 16)] = h10 >> 1
            idx0[pb][pl.ds(2 * B + j * 16, 16)] = h01 >> 1
            idx0[pb][pl.ds(3 * B + j * 16, 16)] = h11 >> 1
            return 0

        lax.fori_loop(0, B // 16, hash_body, 0)

    def fire_gather(b):
        pltpu.async_copy(t8.at[idx0[b]], rows0[b], tsem[b])

    def wait_gather(b):
        pltpu.make_async_copy(t8.at[idx0[b]], rows0[b], tsem[b]).wait()

    def do_blend(c, b):
        def blend_body(j, _):
            row4 = j * 4 + grp4
            xoff = (j >> 5) * 256 + (j & 31) * 4 + grp4
            px = plsc.load_gather(posv[b], [xoff])
            py = plsc.load_gather(posv[b], [xoff + 128])

            def finish(l, acc):
                s1 = acc + _take16(acc, rot1)
                s2 = s1 + _take16(s1, rot2)
                mu = s2 * 0.25
                d = acc - mu
                sq = d * d
                v1 = sq + _take16(sq, rot1)
                v2 = v1 + _take16(v1, rot2)
                inv = _rsqrt(v2 * 0.25 + 1e-5)
                plsc.store_scatter(outv[b], [j * 64 + l * 4 + outpat],
                                   d * inv * lw_splat[l])

            sx = px * 0.25
            sy = py * 0.25
            ixi = sx.astype(jnp.int32)
            iyi = sy.astype(jnp.int32)
            fx = sx - ixi.astype(jnp.float32)
            fy = sy - iyi.astype(jnp.float32)
            gx = 1.0 - fx
            gy = 1.0 - fy
            s = ((ixi ^ iyi) & 1) << 2
            c00 = s + lane4
            c10 = lane4p4 - s
            rv = rows0[b]
            f00 = plsc.load_gather(rv, [0 * B + row4, c00])
            f10 = plsc.load_gather(rv, [1 * B + row4, c10])
            f01 = plsc.load_gather(rv, [2 * B + row4, c10])
            f11 = plsc.load_gather(rv, [3 * B + row4, c00])
            finish(0, (f00 * (gx * gy) + f10 * (fx * gy)
                       + f01 * (gx * fy) + f11 * (fx * fy)))

            for g in range(3):
                ng = GRID_NG[g]
                inv_cs = 0.25 / (2 << g)
                sx = px * inv_cs
                sy = py * inv_cs
                ixi = sx.astype(jnp.int32)
                iyi = sy.astype(jnp.int32)
                fx = sx - ixi.astype(jnp.float32)
                fy = sy - iyi.astype(jnp.float32)
                gx = 1.0 - fx
                gy = 1.0 - fy
                c00 = ((ixi * ng + iyi) << 2) + lane4
                c01 = c00 + 4
                c10 = c00 + 4 * ng
                c11 = c10 + 4
                gr = grids[g]
                f00 = plsc.load_gather(gr, [c00])
                f10 = plsc.load_gather(gr, [c10])
                f01 = plsc.load_gather(gr, [c01])
                f11 = plsc.load_gather(gr, [c11])
                finish(g + 1, (f00 * (gx * gy) + f10 * (fx * gy)
                               + f01 * (gx * fy) + f11 * (fx * fy)))
            return 0

        lax.fori_loop(0, B // 4, blend_body, 0)

    pltpu.sync_copy(pos_slice(0), posv[0])
    do_hash(0, 0)
    fire_gather(0)
    pltpu.async_copy(pos_slice(1), posv[1], psem[1])

    def pair_body(p, _):
        for b in (0, 1):
            i = 2 * p + b

            @pl.when(i + 1 < N_CHUNKS)
            def _(b=b, i=i):
                pltpu.make_async_copy(pos_slice(0), posv[b ^ 1],
                                      psem[b ^ 1]).wait()
                do_hash(i + 1, b ^ 1)
                fire_gather(b ^ 1)

            wait_gather(b)

            @pl.when(i >= 2)
            def _(b=b, i=i):
                pltpu.make_async_copy(outv[b], out_slice(0), osem[b]).wait()

            do_blend(i, b)
            pltpu.async_copy(outv[b], out_slice(i), osem[b])

            @pl.when(i + 2 < N_CHUNKS)
            def _(b=b, i=i):
                pltpu.async_copy(pos_slice(i + 2), posv[b], psem[b])
        return 0

    lax.fori_loop(0, N_CHUNKS // 2, pair_body, 0)
    pltpu.make_async_copy(outv[0], out_slice(0), osem[0]).wait()
    pltpu.make_async_copy(outv[1], out_slice(0), osem[1]).wait()


def kernel(positions, table0, table1, table2, table3, level_weights):
    xs = [t.reshape(NBLK, 128, 4).transpose(0, 2, 1).reshape(-1)
          for t in (table0, table1, table2, table3)]
    t8_0 = _transpose(xs[0])
    pos1d = positions.reshape(N_POS // 128, 128, 2).transpose(0, 2, 1).reshape(-1)
    lw16 = jnp.zeros((16,), jnp.float32).at[:4].set(level_weights)
    out = _mrhg2d(pos1d, t8_0, xs[1], xs[2], xs[3], lw16)
    return out.reshape(N_POS, 16)
